# baseline (speedup 1.0000x reference)
import functools

import jax
import jax.numpy as jnp
from jax import lax
from jax.experimental import pallas as pl
from jax.experimental.pallas import tpu as pltpu
import jax.experimental.pallas.tpu_sc as plsc

N = 10000
E = 320000
D = 128
C = 10

NT = 10240
DGW = 16
NC = 2
NS = 16
NW = NC * NS
BLK = 128
JA = 34
JB = 124
JMX = max(JA, JB)
EPAD = NS * (JA + JB) * BLK

ROWS_PER_TILE = NT // NS
CP = 128
ZROW = NT - CP


def _sc_aggregate(feat16, src_blk, dst_blk, aux):
    mesh = plsc.VectorSubcoreMesh(core_axis_name="c", subcore_axis_name="s")

    @functools.partial(
        pl.kernel,
        out_type=(jax.ShapeDtypeStruct((NC, NT, D), jnp.bfloat16),
                  jax.ShapeDtypeStruct((NC, NT, DGW), jnp.float32)),
        mesh=mesh,
        scratch_types=[
            pltpu.MemorySpace.VMEM_SHARED((NT, D), jnp.bfloat16),
            pltpu.MemorySpace.VMEM_SHARED((NT, DGW), jnp.float32),
            pltpu.MemorySpace.VMEM((JMX, BLK), jnp.int32),
            pltpu.MemorySpace.VMEM((JMX, BLK), jnp.int32),
            pltpu.MemorySpace.VMEM((BLK, D), jnp.bfloat16),
            pltpu.MemorySpace.VMEM((BLK, D), jnp.bfloat16),
            pltpu.MemorySpace.VMEM((BLK, DGW), jnp.float32),
            pltpu.SemaphoreType.DMA,
            pltpu.SemaphoreType.DMA,
        ],
        compiler_params=pltpu.CompilerParams(use_tc_tiling_on_sc=False),
    )
    def body(feat_hbm, src_hbm, dst_hbm, aux_hbm, acc_out, deg_out,
             acc_sh, deg_sh, src_v, dst_v, rows_a, rows_b, ones_v,
             sem_a, sem_b):
        cid = lax.axis_index("c")
        sid = lax.axis_index("s")
        wid = cid * NS + sid
        row0 = sid * ROWS_PER_TILE

        pltpu.sync_copy(feat_hbm.at[pl.ds(ZROW, CP)], rows_a)
        pltpu.sync_copy(aux_hbm.at[pl.ds(BLK, CP)], ones_v)
        for i in range(ROWS_PER_TILE // CP):
            r = row0 + i * CP
            pltpu.sync_copy(rows_a, acc_sh.at[pl.ds(r, CP)])
            pltpu.sync_copy(ones_v, deg_sh.at[pl.ds(r, CP)])

        pltpu.sync_copy(aux_hbm.at[pl.ds(0, BLK)], ones_v)
        pltpu.sync_copy(src_hbm.at[wid], src_v)
        pltpu.sync_copy(dst_hbm.at[wid], dst_v)

        plsc.subcore_barrier()

        nsteps = jnp.where(cid == 0, JA // 2, JB // 2)
        pltpu.async_copy(feat_hbm.at[src_v.at[0]], rows_a, sem_a)

        def step(t, c2):
            j = t * 2
            pltpu.make_async_copy(feat_hbm.at[src_v.at[j]], rows_a, sem_a).wait()
            pltpu.async_copy(feat_hbm.at[src_v.at[j + 1]], rows_b, sem_b)
            pltpu.sync_copy(rows_a, acc_sh.at[dst_v.at[j]], add=True)
            pltpu.sync_copy(ones_v, deg_sh.at[dst_v.at[j]], add=True)
            pltpu.make_async_copy(feat_hbm.at[src_v.at[j + 1]], rows_b, sem_b).wait()

            @pl.when(t + 1 < nsteps)
            def _next():
                pltpu.async_copy(feat_hbm.at[src_v.at[j + 2]], rows_a, sem_a)

            pltpu.sync_copy(rows_b, acc_sh.at[dst_v.at[j + 1]], add=True)
            pltpu.sync_copy(ones_v, deg_sh.at[dst_v.at[j + 1]], add=True)
            return c2

        lax.fori_loop(0, nsteps, step, 0)

        plsc.subcore_barrier()

        for i in range(ROWS_PER_TILE // CP):
            r = row0 + i * CP
            pltpu.sync_copy(acc_sh.at[pl.ds(r, CP)], rows_a)
            pltpu.sync_copy(rows_a, acc_out.at[cid, pl.ds(r, CP)])
            pltpu.sync_copy(deg_sh.at[pl.ds(r, CP)], ones_v)
            pltpu.sync_copy(ones_v, deg_out.at[cid, pl.ds(r, CP)])

    return body(feat16, src_blk, dst_blk, aux)


ROWB = 1000


def _tc_body(x_ref, p_ref, dg_ref, w_ref, wc_ref, bc_ref, out_ref, acc_ref):
    i = pl.program_id(0)

    @pl.when(i == 0)
    def _init():
        acc_ref[...] = jnp.zeros_like(acc_ref)

    x = x_ref[...]
    s = p_ref[0].astype(jnp.float32) + p_ref[1].astype(jnp.float32)
    deg = dg_ref[0, :, 0:1] + dg_ref[1, :, 0:1]
    neigh = s / jnp.clip(deg, 1.0, None)
    w = w_ref[...]
    h = x @ w[:D] + neigh @ w[D:]
    h = jnp.maximum(h, 0.0)
    acc_ref[...] += jnp.sum(h, axis=0, keepdims=True)

    @pl.when(i == pl.num_programs(0) - 1)
    def _fin():
        ge = acc_ref[...]
        scores = lax.dot_general(ge, wc_ref[...], (((1,), (1,)), ((), ())))
        out_ref[...] = scores + bc_ref[...]


def _tc_readout(x, acc, deg, W, Wc, bc2):
    grid = (N // ROWB,)
    return pl.pallas_call(
        _tc_body,
        grid=grid,
        in_specs=[
            pl.BlockSpec((ROWB, D), lambda i: (i, 0)),
            pl.BlockSpec((NC, ROWB, D), lambda i: (0, i, 0)),
            pl.BlockSpec((NC, ROWB, DGW), lambda i: (0, i, 0)),
            pl.BlockSpec((2 * D, D), lambda i: (0, 0)),
            pl.BlockSpec((C, D), lambda i: (0, 0)),
            pl.BlockSpec((1, C), lambda i: (0, 0)),
        ],
        out_specs=pl.BlockSpec((1, C), lambda i: (0, 0)),
        out_shape=jax.ShapeDtypeStruct((1, C), jnp.float32),
        scratch_shapes=[pltpu.VMEM((1, D), jnp.float32)],
    )(x, acc, deg, W, Wc, bc2)


def kernel(features, edge_index, W, Wc, bc):
    f32 = jnp.float32
    feat16 = jnp.concatenate(
        [features.astype(jnp.bfloat16),
         jnp.zeros((NT - N, D), jnp.bfloat16)], axis=0)

    aux = jnp.concatenate(
        [jnp.full((BLK, 1), 1.0, f32),
         jnp.zeros((BLK, DGW - 1), f32)], axis=1)
    aux = jnp.concatenate([aux, jnp.zeros((CP, DGW), f32)], axis=0)

    pad = EPAD - E
    src = jnp.concatenate([edge_index[0], jnp.full((pad,), N, jnp.int32)])
    dst = jnp.concatenate([edge_index[1], jnp.full((pad,), N, jnp.int32)])

    def blockify(flat):
        cut = NS * JA * BLK
        a = flat[:cut].reshape(NS, JA, BLK)
        a = jnp.pad(a, ((0, 0), (0, JMX - JA), (0, 0)))
        b = flat[cut:].reshape(NS, JB, BLK)
        return jnp.concatenate([a, b], axis=0)

    src_blk = blockify(src)
    dst_blk = blockify(dst)

    acc, deg = _sc_aggregate(feat16, src_blk, dst_blk, aux)
    scores = _tc_readout(features, acc, deg, W, Wc, bc.reshape(1, C))
    return scores


# speedup vs baseline: 1.3190x; 1.3190x over previous
import functools

import jax
import jax.numpy as jnp
from jax import lax
from jax.experimental import pallas as pl
from jax.experimental.pallas import tpu as pltpu
import jax.experimental.pallas.tpu_sc as plsc

N = 10000
E = 320000
D = 128
C = 10

NT = 10240
DGW = 16
NC = 2
NS = 16
NW = NC * NS
BLK = 128
JA = 108
JB = 50
JMX = max(JA, JB)
EPAD = NS * (JA + JB) * BLK

ROWS_PER_TILE = NT // NS
CP = 128
ZROW = NT - CP


def _sc_aggregate(feat16, src_blk, dst_blk, aux):
    mesh = plsc.VectorSubcoreMesh(core_axis_name="c", subcore_axis_name="s")

    @functools.partial(
        pl.kernel,
        out_type=(jax.ShapeDtypeStruct((NC, NT, D), jnp.bfloat16),
                  jax.ShapeDtypeStruct((NC, NT, DGW), jnp.float32)),
        mesh=mesh,
        scratch_types=[
            pltpu.MemorySpace.VMEM_SHARED((NT, D), jnp.bfloat16),
            pltpu.MemorySpace.VMEM_SHARED((NT, DGW), jnp.float32),
            pltpu.MemorySpace.VMEM((JMX, BLK), jnp.int32),
            pltpu.MemorySpace.VMEM((JMX, BLK), jnp.int32),
            pltpu.MemorySpace.VMEM((BLK, D), jnp.bfloat16),
            pltpu.MemorySpace.VMEM((BLK, D), jnp.bfloat16),
            pltpu.MemorySpace.VMEM((BLK, DGW), jnp.float32),
            pltpu.SemaphoreType.DMA,
            pltpu.SemaphoreType.DMA,
        ],
        compiler_params=pltpu.CompilerParams(use_tc_tiling_on_sc=False),
    )
    def body(feat_hbm, src_hbm, dst_hbm, aux_hbm, acc_out, deg_out,
             acc_sh, deg_sh, src_v, dst_v, rows_a, rows_b, ones_v,
             sem_a, sem_b):
        cid = lax.axis_index("c")
        sid = lax.axis_index("s")
        wid = cid * NS + sid
        row0 = sid * ROWS_PER_TILE

        pltpu.sync_copy(feat_hbm.at[pl.ds(ZROW, CP)], rows_a)
        pltpu.sync_copy(aux_hbm.at[pl.ds(BLK, CP)], ones_v)
        for i in range(ROWS_PER_TILE // CP):
            r = row0 + i * CP
            pltpu.sync_copy(rows_a, acc_sh.at[pl.ds(r, CP)])
            pltpu.sync_copy(ones_v, deg_sh.at[pl.ds(r, CP)])

        pltpu.sync_copy(aux_hbm.at[pl.ds(0, BLK)], ones_v)
        pltpu.sync_copy(src_hbm.at[wid], src_v)
        pltpu.sync_copy(dst_hbm.at[wid], dst_v)

        plsc.subcore_barrier()

        nsteps = jnp.where(cid == 0, JA // 2, JB // 2)
        pltpu.async_copy(feat_hbm.at[src_v.at[0]], rows_a, sem_a)

        def step(t, c2):
            j = t * 2
            pltpu.make_async_copy(feat_hbm.at[src_v.at[j]], rows_a, sem_a).wait()
            pltpu.async_copy(feat_hbm.at[src_v.at[j + 1]], rows_b, sem_b)
            pltpu.sync_copy(rows_a, acc_sh.at[dst_v.at[j]], add=True)
            pltpu.sync_copy(ones_v, deg_sh.at[dst_v.at[j]], add=True)
            pltpu.make_async_copy(feat_hbm.at[src_v.at[j + 1]], rows_b, sem_b).wait()

            @pl.when(t + 1 < nsteps)
            def _next():
                pltpu.async_copy(feat_hbm.at[src_v.at[j + 2]], rows_a, sem_a)

            pltpu.sync_copy(rows_b, acc_sh.at[dst_v.at[j + 1]], add=True)
            pltpu.sync_copy(ones_v, deg_sh.at[dst_v.at[j + 1]], add=True)
            return c2

        lax.fori_loop(0, nsteps, step, 0)

        plsc.subcore_barrier()

        for i in range(ROWS_PER_TILE // CP):
            r = row0 + i * CP
            pltpu.sync_copy(acc_sh.at[pl.ds(r, CP)], rows_a)
            pltpu.sync_copy(rows_a, acc_out.at[cid, pl.ds(r, CP)])
            pltpu.sync_copy(deg_sh.at[pl.ds(r, CP)], ones_v)
            pltpu.sync_copy(ones_v, deg_out.at[cid, pl.ds(r, CP)])

    return body(feat16, src_blk, dst_blk, aux)


ROWB = 1000


def _tc_body(x_ref, p_ref, dg_ref, w_ref, wc_ref, bc_ref, out_ref, acc_ref):
    i = pl.program_id(0)

    @pl.when(i == 0)
    def _init():
        acc_ref[...] = jnp.zeros_like(acc_ref)

    x = x_ref[...]
    s = p_ref[0].astype(jnp.float32) + p_ref[1].astype(jnp.float32)
    deg = dg_ref[0, :, 0:1] + dg_ref[1, :, 0:1]
    neigh = s / jnp.clip(deg, 1.0, None)
    w = w_ref[...]
    h = x @ w[:D] + neigh @ w[D:]
    h = jnp.maximum(h, 0.0)
    acc_ref[...] += jnp.sum(h, axis=0, keepdims=True)

    @pl.when(i == pl.num_programs(0) - 1)
    def _fin():
        ge = acc_ref[...]
        scores = lax.dot_general(ge, wc_ref[...], (((1,), (1,)), ((), ())))
        out_ref[...] = scores + bc_ref[...]


def _tc_readout(x, acc, deg, W, Wc, bc2):
    grid = (N // ROWB,)
    return pl.pallas_call(
        _tc_body,
        grid=grid,
        in_specs=[
            pl.BlockSpec((ROWB, D), lambda i: (i, 0)),
            pl.BlockSpec((NC, ROWB, D), lambda i: (0, i, 0)),
            pl.BlockSpec((NC, ROWB, DGW), lambda i: (0, i, 0)),
            pl.BlockSpec((2 * D, D), lambda i: (0, 0)),
            pl.BlockSpec((C, D), lambda i: (0, 0)),
            pl.BlockSpec((1, C), lambda i: (0, 0)),
        ],
        out_specs=pl.BlockSpec((1, C), lambda i: (0, 0)),
        out_shape=jax.ShapeDtypeStruct((1, C), jnp.float32),
        scratch_shapes=[pltpu.VMEM((1, D), jnp.float32)],
    )(x, acc, deg, W, Wc, bc2)


def kernel(features, edge_index, W, Wc, bc):
    f32 = jnp.float32
    feat16 = jnp.concatenate(
        [features.astype(jnp.bfloat16),
         jnp.zeros((NT - N, D), jnp.bfloat16)], axis=0)

    aux = jnp.concatenate(
        [jnp.full((BLK, 1), 1.0, f32),
         jnp.zeros((BLK, DGW - 1), f32)], axis=1)
    aux = jnp.concatenate([aux, jnp.zeros((CP, DGW), f32)], axis=0)

    pad = EPAD - E
    src = jnp.concatenate([edge_index[0], jnp.full((pad,), N, jnp.int32)])
    dst = jnp.concatenate([edge_index[1], jnp.full((pad,), N, jnp.int32)])

    def blockify(flat):
        cut = NS * JA * BLK
        a = flat[:cut].reshape(NS, JA, BLK)
        a = jnp.pad(a, ((0, 0), (0, JMX - JA), (0, 0)))
        b = flat[cut:].reshape(NS, JB, BLK)
        b = jnp.pad(b, ((0, 0), (0, JMX - JB), (0, 0)))
        return jnp.concatenate([a, b], axis=0)

    src_blk = blockify(src)
    dst_blk = blockify(dst)

    acc, deg = _sc_aggregate(feat16, src_blk, dst_blk, aux)
    scores = _tc_readout(features, acc, deg, W, Wc, bc.reshape(1, C))
    return scores
